# baseline (speedup 1.0000x reference)
import math

import numpy as np
import jax
import jax.numpy as jnp
from jax import lax
from jax.experimental import pallas as pl
from jax.experimental.pallas import tpu as pltpu

SAMPLE_RATE = 4000
N_FFT = 64
HOP_LENGTH = N_FFT // 2
N_MELS = 16
F_MIN = 0.0
F_MAX = SAMPLE_RATE / 2.0
TOP_DB = 80.0
AMIN = 1e-10
NORM_M = -20.0
NORM_S = 20.0

N_FREQ = N_FFT // 2 + 1
N_PACK = N_FFT
LANES = 128
LOG10_MUL = 10.0 / math.log(10.0)
INV_NORM_S = 1.0 / NORM_S


def _hann_window(n):
    k = np.arange(n, dtype=np.float64)
    return 0.5 * (1.0 - np.cos(2.0 * np.pi * k / n))


def _packed_dft():
    n = np.arange(N_FFT, dtype=np.float64)[:, None]
    k = np.arange(N_FREQ, dtype=np.float64)[None, :]
    ang = 2.0 * np.pi * n * k / N_FFT
    w = _hann_window(N_FFT)[:, None]
    cosw = w * np.cos(ang)
    sinw = -w * np.sin(ang)
    return np.concatenate([cosw, sinw[:, 1:32]], axis=1)


def _edge_maps():
    wfull = _packed_dft()
    m0 = np.zeros((N_FFT, N_FFT))
    for j in range(N_FFT):
        src = 32 - j if j < 32 else j - 32
        m0[src, j] = 1.0
    m1 = np.zeros((N_FFT, N_FFT))
    for j in range(N_FFT):
        src = (32 + j) if j < 32 else (62 - (j - 32))
        m1[src, j] = 1.0
    e0 = (m0 @ wfull).astype(np.float32)
    e1 = (m1 @ wfull).astype(np.float32)
    return e0, e1


def _mel_filterbank():
    def hz_to_mel(f):
        return 2595.0 * np.log10(1.0 + f / 700.0)

    def mel_to_hz(m):
        return 700.0 * (10.0 ** (m / 2595.0) - 1.0)

    all_freqs = np.linspace(0.0, SAMPLE_RATE // 2, N_FREQ)
    m_pts = np.linspace(hz_to_mel(F_MIN), hz_to_mel(F_MAX), N_MELS + 2)
    f_pts = mel_to_hz(m_pts)
    f_diff = f_pts[1:] - f_pts[:-1]
    slopes = f_pts[None, :] - all_freqs[:, None]
    down = -slopes[:, :-2] / f_diff[:-1]
    up = slopes[:, 2:] / f_diff[1:]
    fb = np.maximum(0.0, np.minimum(down, up)).astype(np.float32)
    return np.concatenate([fb.T, fb.T[:, 1:32]], axis=1)


def _selector(bpb, n_super):
    m = bpb * n_super
    s = np.zeros((3 * bpb, m), dtype=np.float32)
    for b in range(bpb):
        s[b, b * n_super:(b + 1) * n_super] = 1.0
        s[bpb + b, b * n_super] = 1.0
        s[2 * bpb + b, (b + 1) * n_super - 1] = 1.0
    return s


def _make_kernel(bpb, n_super, n_rows, n_frames, T):
    m125 = bpb * n_super
    m = bpb * n_rows
    inv_nm1 = 1.0 / (T - 1)
    inv_t = 1.0 / T

    def _body(x_ref, wsup_ref, sel_ref, e0_ref, e1_ref, fbt_ref, perm_ref,
              expand_ref, out_ref):
        xs = x_ref[...].reshape(m125, LANES)
        p_all = jnp.dot(xs, wsup_ref[...],
                        preferred_element_type=jnp.float32)
        z0 = p_all[:, 0:64]
        z1 = p_all[:, 64:128]
        z2 = p_all[:, 128:192]
        z3 = p_all[:, 192:256] + jnp.concatenate(
            [p_all[1:, 256:320], jnp.zeros((1, 64), jnp.float32)],
            axis=0)

        g = jnp.dot(sel_ref[...], xs,
                    preferred_element_type=jnp.float32)
        gsq = jnp.dot(sel_ref[:bpb, :], xs * xs,
                      preferred_element_type=jnp.float32)
        s1 = jnp.sum(g[:bpb], axis=1, keepdims=True)
        s2s = jnp.sum(gsq, axis=1, keepdims=True)
        var = (s2s - s1 * s1 * inv_t) * inv_nm1
        scale2 = 1.0 / var

        y0 = jnp.dot(g[bpb:2 * bpb, :N_FFT], e0_ref[...],
                     preferred_element_type=jnp.float32)
        y1 = jnp.dot(g[2 * bpb:3 * bpb, N_FFT:], e1_ref[...],
                     preferred_element_type=jnp.float32)

        pe = jnp.concatenate([y0, y1], axis=0)
        pe = pe * pe
        mels = [jnp.einsum('mf,nf->mn', fbt_ref[...], z * z,
                           preferred_element_type=jnp.float32)
                for z in (z0, z1, z2, z3)]
        mel_e = jnp.einsum('mf,nf->mn', fbt_ref[...], pe,
                           preferred_element_type=jnp.float32)

        catg = jnp.concatenate(
            [jnp.concatenate(
                [mk[:, b * n_super:(b + 1) * n_super] for mk in mels]
                + [mel_e[:, b:b + 1], mel_e[:, bpb + b:bpb + b + 1]],
                axis=1)
             for b in range(bpb)], axis=0)
        cat = jnp.dot(catg, perm_ref[...],
                      preferred_element_type=jnp.float32)
        s2_sub = jnp.dot(expand_ref[...], scale2,
                         preferred_element_type=jnp.float32)
        db = LOG10_MUL * jnp.log(jnp.maximum(cat * s2_sub, AMIN))
        rmax = jnp.max(db[:, :n_frames], axis=1, keepdims=True)
        lows = []
        for b in range(bpb):
            lo = jnp.max(rmax[N_MELS * b:N_MELS * (b + 1)]) - TOP_DB
            lows.append(jnp.full((N_MELS, 1), lo, jnp.float32))
        lo_sub = jnp.concatenate(lows, axis=0)
        res = (jnp.maximum(db[:, :n_frames], lo_sub) - NORM_M) * INV_NORM_S
        for b in range(bpb):
            out_ref[b] = res[N_MELS * b:N_MELS * (b + 1)]
    return _body


def kernel(x):
    B, T = x.shape
    x = x.astype(jnp.float32)
    n_super = T // LANES
    n_rows = T // HOP_LENGTH
    n_frames = T // HOP_LENGTH + 1

    xs = x

    wfull = _packed_dft().astype(np.float32)
    wsplit = np.concatenate([wfull[:HOP_LENGTH], wfull[HOP_LENGTH:]],
                            axis=1)
    wsup = np.zeros((LANES, 5 * N_PACK), dtype=np.float32)
    for k in range(3):
        wsup[32 * k:32 * k + N_FFT, N_PACK * k:N_PACK * (k + 1)] = wfull
    wsup[96:128, 3 * N_PACK:4 * N_PACK] = wfull[:HOP_LENGTH]
    wsup[0:32, 4 * N_PACK:5 * N_PACK] = wfull[HOP_LENGTH:]
    e0, e1 = _edge_maps()
    fbt = _mel_filterbank()

    bpb = 16
    while B % bpb:
        bpb //= 2
    num_blocks = max(B // bpb, 1)
    sel = _selector(bpb, n_super)
    n_out = ((n_frames + 127) // 128) * 128
    perm = np.zeros((4 * n_super + 2, n_out), dtype=np.float32)
    for k in range(4):
        for r in range(n_super):
            f = 4 * r + k + 1
            perm[n_super * k + r, f if f < n_frames - 1 else n_out - 1] = 1.0
    perm[4 * n_super, 0] = 1.0
    perm[4 * n_super + 1, n_frames - 1] = 1.0
    expand = np.zeros((N_MELS * bpb, bpb), dtype=np.float32)
    for b in range(bpb):
        expand[N_MELS * b:N_MELS * (b + 1), b] = 1.0

    flops = (2 * B * n_rows * HOP_LENGTH * 2 * N_PACK
             + 2 * B * n_rows * N_PACK * N_MELS
             + 2 * B * T * 4
             + 6 * B * n_rows * N_PACK)
    bytes_accessed = (B * T * 4 + B * N_MELS * n_frames * 4
                      + (HOP_LENGTH * 2 * N_PACK + 3 * bpb * bpb * n_super
                         + 2 * N_FFT * N_PACK + N_MELS * N_PACK) * 4)

    out2 = pl.pallas_call(
        _make_kernel(bpb, n_super, n_rows, n_frames, T),
        out_shape=jax.ShapeDtypeStruct((B, N_MELS, n_frames), jnp.float32),
        grid=(num_blocks,),
        in_specs=[
            pl.BlockSpec((bpb, T), lambda i: (i, 0)),
            pl.BlockSpec((LANES, 5 * N_PACK), lambda i: (0, 0)),
            pl.BlockSpec((3 * bpb, bpb * n_super), lambda i: (0, 0)),
            pl.BlockSpec((N_FFT, N_PACK), lambda i: (0, 0)),
            pl.BlockSpec((N_FFT, N_PACK), lambda i: (0, 0)),
            pl.BlockSpec((N_MELS, N_PACK), lambda i: (0, 0)),
            pl.BlockSpec((4 * n_super + 2, n_out), lambda i: (0, 0)),
            pl.BlockSpec((N_MELS * bpb, bpb), lambda i: (0, 0)),
        ],
        out_specs=pl.BlockSpec((bpb, N_MELS, n_frames), lambda i: (i, 0, 0)),
        compiler_params=pltpu.CompilerParams(
            dimension_semantics=("parallel",)),
        cost_estimate=pl.CostEstimate(
            flops=int(flops),
            transcendentals=int(B * n_frames * N_MELS),
            bytes_accessed=int(bytes_accessed)),
    )(xs, jnp.asarray(wsup), jnp.asarray(sel), jnp.asarray(e0),
      jnp.asarray(e1), jnp.asarray(fbt), jnp.asarray(perm),
      jnp.asarray(expand))

    return out2


# speedup vs baseline: 1.0989x; 1.0989x over previous
import math

import numpy as np
import jax
import jax.numpy as jnp
from jax.experimental import pallas as pl
from jax.experimental.pallas import tpu as pltpu

SAMPLE_RATE = 4000
N_FFT = 64
HOP_LENGTH = N_FFT // 2
N_MELS = 16
F_MIN = 0.0
F_MAX = SAMPLE_RATE / 2.0
TOP_DB = 80.0
AMIN = 1e-10
NORM_M = -20.0
NORM_S = 20.0

N_FREQ = N_FFT // 2 + 1
N_PACK = N_FFT
LANES = 128
LOG10_MUL = 10.0 / math.log(10.0)
INV_NORM_S = 1.0 / NORM_S


def _hann_window(n):
    k = np.arange(n, dtype=np.float64)
    return 0.5 * (1.0 - np.cos(2.0 * np.pi * k / n))


def _packed_dft():
    n = np.arange(N_FFT, dtype=np.float64)[:, None]
    k = np.arange(N_FREQ, dtype=np.float64)[None, :]
    ang = 2.0 * np.pi * n * k / N_FFT
    w = _hann_window(N_FFT)[:, None]
    cosw = w * np.cos(ang)
    sinw = -w * np.sin(ang)
    return np.concatenate([cosw, sinw[:, 1:32]], axis=1)


def _edge_maps():
    wfull = _packed_dft()
    m0 = np.zeros((N_FFT, N_FFT))
    for j in range(N_FFT):
        src = 32 - j if j < 32 else j - 32
        m0[src, j] = 1.0
    m1 = np.zeros((N_FFT, N_FFT))
    for j in range(N_FFT):
        src = (32 + j) if j < 32 else (62 - (j - 32))
        m1[src, j] = 1.0
    e0 = (m0 @ wfull).astype(np.float32)
    e1 = (m1 @ wfull).astype(np.float32)
    return e0, e1


def _mel_filterbank():
    def hz_to_mel(f):
        return 2595.0 * np.log10(1.0 + f / 700.0)

    def mel_to_hz(m):
        return 700.0 * (10.0 ** (m / 2595.0) - 1.0)

    all_freqs = np.linspace(0.0, SAMPLE_RATE // 2, N_FREQ)
    m_pts = np.linspace(hz_to_mel(F_MIN), hz_to_mel(F_MAX), N_MELS + 2)
    f_pts = mel_to_hz(m_pts)
    f_diff = f_pts[1:] - f_pts[:-1]
    slopes = f_pts[None, :] - all_freqs[:, None]
    down = -slopes[:, :-2] / f_diff[:-1]
    up = slopes[:, 2:] / f_diff[1:]
    fb = np.maximum(0.0, np.minimum(down, up)).astype(np.float32)
    return np.concatenate([fb.T, fb.T[:, 1:32]], axis=1)


def _selector(bpb, n_super):
    m = bpb * n_super
    s = np.zeros((3 * bpb, m), dtype=np.float32)
    for b in range(bpb):
        s[b, b * n_super:(b + 1) * n_super] = 1.0
        s[bpb + b, b * n_super] = 1.0
        s[2 * bpb + b, (b + 1) * n_super - 1] = 1.0
    return s


def _make_kernel(bpb, n_super, n_rows, n_frames, T):
    m125 = bpb * n_super

    inv_nm1 = 1.0 / (T - 1)
    inv_t = 1.0 / T

    def _body(x_ref, wsup_ref, sel_ref, e0_ref, e1_ref, fbt_ref, perm_ref,
              expand_ref, out_ref):
        xs = x_ref[...].reshape(m125, LANES)
        p_all = jnp.dot(xs, wsup_ref[...],
                        preferred_element_type=jnp.float32)
        z0 = p_all[:, 0:64]
        z1 = p_all[:, 64:128]
        z2 = p_all[:, 128:192]
        z3 = p_all[:, 192:256] + jnp.concatenate(
            [p_all[1:, 256:320], jnp.zeros((1, 64), jnp.float32)],
            axis=0)

        g = jnp.dot(sel_ref[...], xs,
                    preferred_element_type=jnp.float32)
        gsq = jnp.dot(sel_ref[:bpb, :], xs * xs,
                      preferred_element_type=jnp.float32)
        s1 = jnp.sum(g[:bpb], axis=1, keepdims=True)
        s2s = jnp.sum(gsq, axis=1, keepdims=True)
        var = (s2s - s1 * s1 * inv_t) * inv_nm1
        scale2 = 1.0 / var

        y0 = jnp.dot(g[bpb:2 * bpb, :N_FFT], e0_ref[...],
                     preferred_element_type=jnp.float32)
        y1 = jnp.dot(g[2 * bpb:3 * bpb, N_FFT:], e1_ref[...],
                     preferred_element_type=jnp.float32)

        pe = jnp.concatenate([y0, y1], axis=0)
        pe = pe * pe
        mels = [jnp.einsum('mf,nf->mn', fbt_ref[...], z * z,
                           preferred_element_type=jnp.float32)
                for z in (z0, z1, z2, z3)]
        mel_e = jnp.einsum('mf,nf->mn', fbt_ref[...], pe,
                           preferred_element_type=jnp.float32)

        catg = jnp.concatenate(
            [jnp.concatenate(
                [mk[:, b * n_super:(b + 1) * n_super] for mk in mels]
                + [mel_e[:, b:b + 1], mel_e[:, bpb + b:bpb + b + 1]],
                axis=1)
             for b in range(bpb)], axis=0)
        cat = jnp.dot(catg, perm_ref[...],
                      preferred_element_type=jnp.float32)
        s2_sub = jnp.dot(expand_ref[...], scale2,
                         preferred_element_type=jnp.float32)
        db = LOG10_MUL * jnp.log(jnp.maximum(cat * s2_sub, AMIN))
        rmax = jnp.max(db[:, :n_frames], axis=1, keepdims=True)
        lows = []
        for b in range(bpb):
            lo = jnp.max(rmax[N_MELS * b:N_MELS * (b + 1)]) - TOP_DB
            lows.append(jnp.full((N_MELS, 1), lo, jnp.float32))
        lo_sub = jnp.concatenate(lows, axis=0)
        res = (jnp.maximum(db[:, :n_frames], lo_sub) - NORM_M) * INV_NORM_S
        for b in range(bpb):
            out_ref[b] = res[N_MELS * b:N_MELS * (b + 1)]
    return _body


def kernel(x):
    B, T = x.shape
    x = x.astype(jnp.float32)
    n_super = T // LANES
    n_rows = T // HOP_LENGTH
    n_frames = T // HOP_LENGTH + 1

    wfull = _packed_dft().astype(np.float32)
    wsup = np.zeros((LANES, 5 * N_PACK), dtype=np.float32)
    for k in range(3):
        wsup[32 * k:32 * k + N_FFT, N_PACK * k:N_PACK * (k + 1)] = wfull
    wsup[96:128, 3 * N_PACK:4 * N_PACK] = wfull[:HOP_LENGTH]
    wsup[0:32, 4 * N_PACK:5 * N_PACK] = wfull[HOP_LENGTH:]
    e0, e1 = _edge_maps()
    fbt = _mel_filterbank()

    bpb = 32
    while B % bpb:
        bpb //= 2
    num_blocks = max(B // bpb, 1)
    sel = _selector(bpb, n_super)
    n_out = ((n_frames + 127) // 128) * 128
    perm = np.zeros((4 * n_super + 2, n_out), dtype=np.float32)
    for k in range(4):
        for r in range(n_super):
            f = 4 * r + k + 1
            perm[n_super * k + r, f if f < n_frames - 1 else n_out - 1] = 1.0
    perm[4 * n_super, 0] = 1.0
    perm[4 * n_super + 1, n_frames - 1] = 1.0
    expand = np.zeros((N_MELS * bpb, bpb), dtype=np.float32)
    for b in range(bpb):
        expand[N_MELS * b:N_MELS * (b + 1), b] = 1.0

    flops = (2 * B * n_rows * HOP_LENGTH * 2 * N_PACK
             + 2 * B * n_rows * N_PACK * N_MELS
             + 2 * B * T * 4
             + 6 * B * n_rows * N_PACK)
    bytes_accessed = (B * T * 4 + B * N_MELS * n_frames * 4
                      + (HOP_LENGTH * 2 * N_PACK + 3 * bpb * bpb * n_super
                         + 2 * N_FFT * N_PACK + N_MELS * N_PACK) * 4)

    out2 = pl.pallas_call(
        _make_kernel(bpb, n_super, n_rows, n_frames, T),
        out_shape=jax.ShapeDtypeStruct((B, N_MELS, n_frames), jnp.float32),
        grid=(num_blocks,),
        in_specs=[
            pl.BlockSpec((bpb, T), lambda i: (i, 0)),
            pl.BlockSpec((LANES, 5 * N_PACK), lambda i: (0, 0)),
            pl.BlockSpec((3 * bpb, bpb * n_super), lambda i: (0, 0)),
            pl.BlockSpec((N_FFT, N_PACK), lambda i: (0, 0)),
            pl.BlockSpec((N_FFT, N_PACK), lambda i: (0, 0)),
            pl.BlockSpec((N_MELS, N_PACK), lambda i: (0, 0)),
            pl.BlockSpec((4 * n_super + 2, n_out), lambda i: (0, 0)),
            pl.BlockSpec((N_MELS * bpb, bpb), lambda i: (0, 0)),
        ],
        out_specs=pl.BlockSpec((bpb, N_MELS, n_frames), lambda i: (i, 0, 0)),
        compiler_params=pltpu.CompilerParams(
            dimension_semantics=("parallel",)),
        cost_estimate=pl.CostEstimate(
            flops=int(flops),
            transcendentals=int(B * n_frames * N_MELS),
            bytes_accessed=int(bytes_accessed)),
    )(x, jnp.asarray(wsup), jnp.asarray(sel), jnp.asarray(e0),
      jnp.asarray(e1), jnp.asarray(fbt), jnp.asarray(perm),
      jnp.asarray(expand))

    return out2
